# Initial kernel scaffold; baseline (speedup 1.0000x reference)
import jax
import jax.numpy as jnp
from jax.experimental import pallas as pl


def kernel(x, edge_index, edge_weight, edge_attr, mlp_w1, mlp_b1, mlp_w2, mlp_b2, lin1_w, lin2_w, lin2_b):
    raise NotImplementedError("write your pallas kernel here")


# speedup vs baseline: 2.2771x; 2.2771x over previous
import functools

import jax
import jax.numpy as jnp
from jax import lax
from jax.experimental import pallas as pl
from jax.experimental.pallas import tpu as pltpu
from jax.experimental.pallas import tpu_sc as plsc

N = 10000
E = 320000
H = 128
R = 64
F = 128
CUTOFF = 5.0
LOG2 = 0.6931471805599453

NC = 2
NS = 16
NW = NC * NS
EPW = E // NW
C = 80
NCHUNK = EPW // C
NP = 10240
RPT = NP // NS
ZB = 128


def _ssp(v):
    return jnp.maximum(v, 0.0) + jnp.log1p(jnp.exp(-jnp.abs(v))) - LOG2


EB = 3200

def _edge_filter_body(ea_ref, ew_ref, w1_ref, b1_ref, w2_ref, b2_ref, out_ref):
    ew = ew_ref[...]
    d = jnp.sqrt(jnp.sum(ew * ew, axis=1, keepdims=True))
    u = d * (1.0 / CUTOFF)
    env = jnp.where(u < 1.0, 1.0 - 3.0 * u * u + 2.0 * u * u * u, 0.0)
    h1 = _ssp(jnp.dot(ea_ref[...], w1_ref[...],
                      preferred_element_type=jnp.float32) + b1_ref[...])
    out_ref[...] = (jnp.dot(h1, w2_ref[...],
                            preferred_element_type=jnp.float32)
                    + b2_ref[...]) * env


def _edge_filter(edge_attr, edge_weight, w1, b1, w2, b2):
    grid = E // EB
    return pl.pallas_call(
        _edge_filter_body,
        grid=(grid,),
        in_specs=[
            pl.BlockSpec((EB, R), lambda g: (g, 0)),
            pl.BlockSpec((EB, 3), lambda g: (g, 0)),
            pl.BlockSpec((R, F), lambda g: (0, 0)),
            pl.BlockSpec((1, F), lambda g: (0, 0)),
            pl.BlockSpec((F, F), lambda g: (0, 0)),
            pl.BlockSpec((1, F), lambda g: (0, 0)),
        ],
        out_specs=pl.BlockSpec((EB, F), lambda g: (g, 0)),
        out_shape=jax.ShapeDtypeStruct((E, F), jnp.float32),
    )(edge_attr, edge_weight, w1, b1.reshape(1, F), w2, b2.reshape(1, F))


def _xw_body(x_ref, w_ref, o_ref):
    o_ref[...] = jnp.dot(x_ref[...], w_ref[...],
                         preferred_element_type=jnp.float32)


def _node_transform(x, lin1_w):
    return pl.pallas_call(
        _xw_body,
        out_shape=jax.ShapeDtypeStruct((N, F), jnp.float32),
    )(x, lin1_w)


def _sc_body(w_hbm, xw_hbm, jidx_hbm, iidx_hbm, out_hbm,
             jv, iv, wv, rv, zv, agg_sh, sem_w, sem_g):
    cid = lax.axis_index("c")
    sid = lax.axis_index("s")
    wid = cid * NS + sid
    row0 = sid * RPT

    def _zrow(r, t):
        for q in range(F // 16):
            zv[r, pl.ds(q * 16, 16)] = jnp.zeros((16,), jnp.float32)
        return t
    lax.fori_loop(0, ZB, _zrow, 0)
    for t in range(RPT // ZB):
        pltpu.sync_copy(zv, agg_sh.at[pl.ds(row0 + t * ZB, ZB)])
    plsc.subcore_barrier()

    def _step(k, t):
        base = wid * EPW + k * C
        pltpu.sync_copy(jidx_hbm.at[pl.ds(base, C)], jv)
        pltpu.sync_copy(iidx_hbm.at[pl.ds(base, C)], iv)
        dw = pltpu.async_copy(w_hbm.at[pl.ds(base, C)], wv, sem_w)
        dg = pltpu.async_copy(xw_hbm.at[jv], rv, sem_g)
        dw.wait()
        dg.wait()

        def _mul(r, t2):
            for q in range(F // 16):
                s = pl.ds(q * 16, 16)
                wv[r, s] = wv[r, s] * rv[r, s]
            return t2
        lax.fori_loop(0, C, _mul, 0)
        pltpu.sync_copy(wv, agg_sh.at[iv], add=True)
        return t

    lax.fori_loop(0, NCHUNK, _step, 0)
    plsc.subcore_barrier()

    pltpu.sync_copy(agg_sh.at[pl.ds(row0, RPT)],
                    out_hbm.at[cid, pl.ds(row0, RPT)])


def _sc_aggregate(w_edges, xw, jidx, iidx):
    mesh = plsc.VectorSubcoreMesh(core_axis_name="c", subcore_axis_name="s",
                                  num_cores=NC, num_subcores=NS)
    kern = functools.partial(
        pl.kernel,
        out_type=jax.ShapeDtypeStruct((NC, NP, F), jnp.float32),
        mesh=mesh,
        scratch_types=[
            pltpu.VMEM((C,), jnp.int32),
            pltpu.VMEM((C,), jnp.int32),
            pltpu.VMEM((C, F), jnp.float32),
            pltpu.VMEM((C, F), jnp.float32),
            pltpu.VMEM((ZB, F), jnp.float32),
            pltpu.VMEM_SHARED((NP, F), jnp.float32),
            pltpu.SemaphoreType.DMA,
            pltpu.SemaphoreType.DMA,
        ],
    )(_sc_body)
    return kern(w_edges, xw, jidx, iidx)


NB = 2000

def _final_body(x_ref, p_ref, w_ref, b_ref, o_ref):
    agg = p_ref[0] + p_ref[1]
    h = jnp.dot(agg, w_ref[...], preferred_element_type=jnp.float32) + b_ref[...]
    o_ref[...] = x_ref[...] + _ssp(h)


def _final(x, partials, lin2_w, lin2_b):
    grid = N // NB
    return pl.pallas_call(
        _final_body,
        grid=(grid,),
        in_specs=[
            pl.BlockSpec((NB, H), lambda g: (g, 0)),
            pl.BlockSpec((NC, NB, F), lambda g: (0, g, 0)),
            pl.BlockSpec((F, H), lambda g: (0, 0)),
            pl.BlockSpec((1, H), lambda g: (0, 0)),
        ],
        out_specs=pl.BlockSpec((NB, H), lambda g: (g, 0)),
        out_shape=jax.ShapeDtypeStruct((N, H), jnp.float32),
    )(x, partials, lin2_w, lin2_b.reshape(1, H))


def kernel(x, edge_index, edge_weight, edge_attr,
           mlp_w1, mlp_b1, mlp_w2, mlp_b2, lin1_w, lin2_w, lin2_b):
    w_edges = _edge_filter(edge_attr, edge_weight, mlp_w1, mlp_b1, mlp_w2, mlp_b2)
    xw = _node_transform(x, lin1_w)
    iidx = edge_index[0]
    jidx = edge_index[1]
    partials = _sc_aggregate(w_edges, xw, jidx, iidx)
    return _final(x, partials, lin2_w, lin2_b)
